# baseline (speedup 1.0000x reference)
import jax
import jax.numpy as jnp
import numpy as np
from functools import partial
from jax.experimental import pallas as pl

RATIO = 0.25
K = 32


def _res_mlp(h, w1, b1, w2, b2, ws, bs):
    out = jnp.maximum(h @ w1 + b1, 0.0) @ w2 + b2
    return out + (h @ ws + bs)


@partial(jax.jit, static_argnums=1)
def _fps(pos, n_samples):
    dists = jnp.sum((pos - pos[0]) ** 2, axis=1)
    samples = jnp.zeros((n_samples,), dtype=jnp.int32)

    def body(i, st):
        d_min, smp = st
        last = smp[i - 1]
        d = jnp.sum((pos - pos[last]) ** 2, axis=1)
        d_min = jnp.minimum(d_min, d)
        nxt = jnp.argmax(d_min).astype(jnp.int32)
        return d_min, smp.at[i].set(nxt)

    _, samples = jax.lax.fori_loop(1, n_samples, body, (dists, samples))
    return samples


def _knn(pos, q, k, chunk=500):
    nq = q.shape[0]
    pad = (-nq) % chunk
    if pad:
        q = jnp.concatenate([q, jnp.zeros((pad, 3), q.dtype)], axis=0)
    qc = q.reshape(-1, chunk, 3)
    p2 = jnp.sum(pos ** 2, axis=1)

    def f(qb):
        d = jnp.sum(qb ** 2, axis=1)[:, None] + p2[None, :] - 2.0 * (qb @ pos.T)
        _, nb = jax.lax.top_k(-d, k)
        return nb

    nb = jax.lax.map(f, qc).reshape(-1, k)[:nq]
    return nb.astype(jnp.int32)


def _relu_kernel(m_ref, o_ref):
    o_ref[...] = jnp.maximum(m_ref[...], 0.0)


def _pallas_relu(m):
    rows = m.shape[0]
    blk = rows if rows <= 20000 else 10000
    assert rows % blk == 0
    return pl.pallas_call(
        _relu_kernel,
        grid=(rows // blk,),
        in_specs=[pl.BlockSpec((blk, m.shape[1]), lambda i: (i, 0))],
        out_specs=pl.BlockSpec((blk, m.shape[1]), lambda i: (i, 0)),
        out_shape=jax.ShapeDtypeStruct(m.shape, m.dtype),
    )(m)


def kernel(x, pos, batch, w1, b1, w2, b2, ws, bs):
    n = pos.shape[0]
    n_samples = int(np.ceil(RATIO * n))
    idx = _fps(pos, n_samples)
    q = pos[idx]
    nbr = _knn(pos, q, K)
    row = jnp.repeat(jnp.arange(n_samples, dtype=jnp.int32), K)
    col = nbr.reshape(-1)
    h = jnp.concatenate([x[col], pos[col] - q[row]], axis=-1)
    msg = _pallas_relu(_res_mlp(h, w1, b1, w2, b2, ws, bs))
    msg = jnp.where((col == row)[:, None], -jnp.inf, msg)
    i = jnp.arange(n_samples, dtype=jnp.int32)
    h_sl = jnp.concatenate([x[:n_samples], pos[:n_samples] - q], axis=-1)
    msg_sl = _pallas_relu(_res_mlp(h_sl, w1, b1, w2, b2, ws, bs))
    out = jax.ops.segment_max(
        jnp.concatenate([msg, msg_sl], axis=0),
        jnp.concatenate([row, i], axis=0),
        num_segments=n_samples)
    return out, q, batch[idx]


# speedup vs baseline: 3.4874x; 3.4874x over previous
import jax
import jax.numpy as jnp
import numpy as np
from functools import partial
from jax import lax
from jax.experimental import pallas as pl

RATIO = 0.25
K = 32
LANES = 128


def _fps_body(px_ref, py_ref, pz_ref, b_ref, out_ref, *, n, n_samples, rows):
    flat = (lax.broadcasted_iota(jnp.int32, (rows, LANES), 0) * LANES
            + lax.broadcasted_iota(jnp.int32, (rows, LANES), 1))
    valid = flat < n
    px = px_ref[...]
    py = py_ref[...]
    pz = pz_ref[...]
    bt = b_ref[...]
    lane = lax.broadcasted_iota(jnp.int32, (1, LANES), 1)

    def compose(idx_i32, x, y, z, b_i32):
        idxf = lax.bitcast_convert_type(idx_i32, jnp.float32)
        bf = lax.bitcast_convert_type(b_i32, jnp.float32)
        return jnp.where(
            lane == 0, idxf,
            jnp.where(lane == 1, x,
                      jnp.where(lane == 2, y,
                                jnp.where(lane == 3, z,
                                          jnp.where(lane == 4, bf, 0.0)))))

    m0 = flat == 0
    lx = jnp.sum(jnp.where(m0, px, 0.0))
    ly = jnp.sum(jnp.where(m0, py, 0.0))
    lz = jnp.sum(jnp.where(m0, pz, 0.0))
    b0 = jnp.sum(jnp.where(m0, bt, 0))
    out_ref[0:1, :] = compose(jnp.int32(0), lx, ly, lz, b0)

    d_min0 = jnp.where(valid, jnp.float32(np.inf), jnp.float32(-np.inf))

    def body(i, carry):
        d_min, lx, ly, lz = carry
        d = ((px - lx) ** 2 + (pz - lz) ** 2) + (py - ly) ** 2
        d_min = jnp.minimum(d_min, d)
        m = jnp.max(d_min)
        nxt = jnp.min(jnp.where(d_min == m, flat, jnp.int32(2 ** 30)))
        mk = flat == nxt
        nx = jnp.sum(jnp.where(mk, px, 0.0))
        ny = jnp.sum(jnp.where(mk, py, 0.0))
        nz = jnp.sum(jnp.where(mk, pz, 0.0))
        nb = jnp.sum(jnp.where(mk, bt, 0))
        out_ref[pl.ds(i, 1), :] = compose(nxt, nx, ny, nz, nb)
        return d_min, nx, ny, nz

    lax.fori_loop(1, n_samples, body, (d_min0, lx, ly, lz))


def _fps_pallas(pos, batch, n_samples):
    n = pos.shape[0]
    rows = (n + LANES - 1) // LANES
    rows = ((rows + 7) // 8) * 8
    pn = rows * LANES
    pad = pn - n
    px = jnp.pad(pos[:, 0], (0, pad)).reshape(rows, LANES)
    py = jnp.pad(pos[:, 1], (0, pad)).reshape(rows, LANES)
    pz = jnp.pad(pos[:, 2], (0, pad)).reshape(rows, LANES)
    bt = jnp.pad(batch, (0, pad)).reshape(rows, LANES)
    out = pl.pallas_call(
        partial(_fps_body, n=n, n_samples=n_samples, rows=rows),
        out_shape=jax.ShapeDtypeStruct((n_samples, LANES), jnp.float32),
    )(px, py, pz, bt)
    idx = lax.bitcast_convert_type(out[:, 0], jnp.int32)
    q = out[:, 1:4]
    b_out = lax.bitcast_convert_type(out[:, 4], jnp.int32)
    return idx, q, b_out


def _res_mlp(h, w1, b1, w2, b2, ws, bs):
    out = jnp.maximum(h @ w1 + b1, 0.0) @ w2 + b2
    return out + (h @ ws + bs)


def _knn(pos, q, k, chunk=500):
    nq = q.shape[0]
    qc = q.reshape(-1, chunk, 3)
    p2 = jnp.sum(pos ** 2, axis=1)

    def f(qb):
        d = jnp.sum(qb ** 2, axis=1)[:, None] + p2[None, :] - 2.0 * (qb @ pos.T)
        _, nb = jax.lax.top_k(-d, k)
        return nb

    nb = jax.lax.map(f, qc).reshape(-1, k)[:nq]
    return nb.astype(jnp.int32)


def kernel(x, pos, batch, w1, b1, w2, b2, ws, bs):
    n = pos.shape[0]
    n_samples = int(np.ceil(RATIO * n))
    idx, q, b_out = _fps_pallas(pos, batch, n_samples)
    nbr = _knn(pos, q, K)
    row = jnp.repeat(jnp.arange(n_samples, dtype=jnp.int32), K)
    col = nbr.reshape(-1)
    h = jnp.concatenate([x[col], pos[col] - q[row]], axis=-1)
    msg = jnp.maximum(_res_mlp(h, w1, b1, w2, b2, ws, bs), 0.0)
    msg = jnp.where((col == row)[:, None], -jnp.inf, msg)
    i = jnp.arange(n_samples, dtype=jnp.int32)
    h_sl = jnp.concatenate([x[:n_samples], pos[:n_samples] - q], axis=-1)
    msg_sl = jnp.maximum(_res_mlp(h_sl, w1, b1, w2, b2, ws, bs), 0.0)
    out = jax.ops.segment_max(
        jnp.concatenate([msg, msg_sl], axis=0),
        jnp.concatenate([row, i], axis=0),
        num_segments=n_samples)
    return out, q, b_out
